# baseline (speedup 1.0000x reference)
import functools

import jax
import jax.numpy as jnp
from jax import lax
from jax.experimental import pallas as pl
from jax.experimental.pallas import tpu as pltpu

_MASK_RATIO = 0.5
_L = 512
_D = 64
_ROWS = 512
_CHUNK_ROWS = 128
_NCHUNKS = _ROWS // _CHUNK_ROWS
_NBUF = 3
_BITS_HI = 0x3F800000


def _masked_fill_kernel(bits_ref, xt_ref, out_ref, mask_ref,
                        keep_ref, bufs_ref, ld_sems, st_sems, *, len_keep):
    def start_load(c):
        pltpu.make_async_copy(
            xt_ref.at[pl.ds(c * _CHUNK_ROWS, _CHUNK_ROWS)],
            bufs_ref.at[c % _NBUF], ld_sems.at[c % _NBUF]).start()

    def wait_load(c):
        pltpu.make_async_copy(
            xt_ref.at[pl.ds(c * _CHUNK_ROWS, _CHUNK_ROWS)],
            bufs_ref.at[c % _NBUF], ld_sems.at[c % _NBUF]).wait()

    def start_store(c):
        pltpu.make_async_copy(
            bufs_ref.at[c % _NBUF],
            out_ref.at[pl.ds(c * _CHUNK_ROWS, _CHUNK_ROWS)],
            st_sems.at[c % _NBUF]).start()

    def wait_store(c):
        pltpu.make_async_copy(
            bufs_ref.at[c % _NBUF],
            out_ref.at[pl.ds(c * _CHUNK_ROWS, _CHUNK_ROWS)],
            st_sems.at[c % _NBUF]).wait()

    for c in range(_NBUF):
        start_load(c)

    bits = bits_ref[...]
    rows = bits.shape[0]
    lo = jnp.zeros((rows, 1), jnp.int32)
    hi = jnp.full((rows, 1), _BITS_HI, jnp.int32)

    def body(_, carry):
        lo, hi = carry
        mid = (lo + hi) // 2
        cnt = jnp.sum((bits <= mid).astype(jnp.int32), axis=1, keepdims=True)
        pred = cnt >= len_keep
        hi = jnp.where(pred, mid, hi)
        lo = jnp.where(pred, lo, mid + 1)
        return lo, hi

    lo, hi = lax.fori_loop(0, 31, body, (lo, hi))
    t = lo

    cnt_lt = jnp.sum((bits < t).astype(jnp.float32), axis=1, keepdims=True)
    eq = bits == t

    row_ids = lax.broadcasted_iota(jnp.int32, (_L, _L), 0)
    col_ids = lax.broadcasted_iota(jnp.int32, (_L, _L), 1)
    tri = (row_ids < col_ids).astype(jnp.float32)
    eq_rank = jnp.dot(eq.astype(jnp.float32), tri,
                      preferred_element_type=jnp.float32)

    keep = (bits < t) | (eq & (cnt_lt + eq_rank < float(len_keep)))
    keepf = keep.astype(jnp.float32)
    keep_ref[...] = keepf
    mask_ref[...] = 1.0 - keepf

    for c in range(_NCHUNKS):
        wait_load(c)
        b = c % _NBUF
        for i in range(_CHUNK_ROWS):
            row = c * _CHUNK_ROWS + i
            bufs_ref[b, i] = bufs_ref[b, i] * keep_ref[row:row + 1]
        start_store(c)
        nc = c + _NBUF
        if nc < _NCHUNKS:
            wait_store(nc - _NBUF)
            start_load(nc)
    for c in range(_NCHUNKS - _NBUF, _NCHUNKS):
        wait_store(c)


@jax.jit
def kernel(x):
    bs, nvars, L, D = x.shape
    len_keep = int(L * (1 - _MASK_RATIO))
    rows = bs * nvars

    noise = jax.random.uniform(jax.random.key(42), (bs, nvars, L),
                               dtype=jnp.float32)
    bits = lax.bitcast_convert_type(noise, jnp.int32).reshape(rows, L)

    xt = x.transpose(0, 1, 3, 2).reshape(rows, D, L)

    out, maskf = pl.pallas_call(
        functools.partial(_masked_fill_kernel, len_keep=len_keep),
        in_specs=[
            pl.BlockSpec((rows, L), lambda: (0, 0)),
            pl.BlockSpec(memory_space=pl.ANY),
        ],
        out_specs=[
            pl.BlockSpec(memory_space=pl.ANY),
            pl.BlockSpec((rows, L), lambda: (0, 0)),
        ],
        out_shape=[
            jax.ShapeDtypeStruct((rows, D, L), x.dtype),
            jax.ShapeDtypeStruct((rows, L), jnp.float32),
        ],
        scratch_shapes=[
            pltpu.VMEM((rows, L), jnp.float32),
            pltpu.VMEM((_NBUF, _CHUNK_ROWS, D, L), jnp.float32),
            pltpu.SemaphoreType.DMA((_NBUF,)),
            pltpu.SemaphoreType.DMA((_NBUF,)),
        ],
    )(bits, xt)

    x_mask = out.reshape(bs, nvars, D, L).transpose(0, 1, 3, 2)
    mask = maskf.reshape(bs, nvars, L).astype(bool)
    return (x_mask, mask)


# speedup vs baseline: 1.0048x; 1.0048x over previous
import functools

import jax
import jax.numpy as jnp
from jax import lax
from jax.experimental import pallas as pl
from jax.experimental.pallas import tpu as pltpu

_MASK_RATIO = 0.5
_L = 512
_D = 64
_ROWS = 512
_CHUNK_ROWS = 64
_NCHUNKS = _ROWS // _CHUNK_ROWS
_NBUF = 6
_BITS_HI = 0x3F800000


def _masked_fill_kernel(bits_ref, xt_ref, out_ref, mask_ref,
                        keep_ref, bufs_ref, ld_sems, st_sems, *, len_keep):
    def start_load(c):
        pltpu.make_async_copy(
            xt_ref.at[pl.ds(c * _CHUNK_ROWS, _CHUNK_ROWS)],
            bufs_ref.at[c % _NBUF], ld_sems.at[c % _NBUF]).start()

    def wait_load(c):
        pltpu.make_async_copy(
            xt_ref.at[pl.ds(c * _CHUNK_ROWS, _CHUNK_ROWS)],
            bufs_ref.at[c % _NBUF], ld_sems.at[c % _NBUF]).wait()

    def start_store(c):
        pltpu.make_async_copy(
            bufs_ref.at[c % _NBUF],
            out_ref.at[pl.ds(c * _CHUNK_ROWS, _CHUNK_ROWS)],
            st_sems.at[c % _NBUF]).start()

    def wait_store(c):
        pltpu.make_async_copy(
            bufs_ref.at[c % _NBUF],
            out_ref.at[pl.ds(c * _CHUNK_ROWS, _CHUNK_ROWS)],
            st_sems.at[c % _NBUF]).wait()

    for c in range(_NBUF):
        start_load(c)

    bits = bits_ref[...]
    rows = bits.shape[0]
    lo = jnp.zeros((rows, 1), jnp.int32)
    hi = jnp.full((rows, 1), _BITS_HI, jnp.int32)

    def body(_, carry):
        lo, hi = carry
        mid = (lo + hi) // 2
        cnt = jnp.sum((bits <= mid).astype(jnp.int32), axis=1, keepdims=True)
        pred = cnt >= len_keep
        hi = jnp.where(pred, mid, hi)
        lo = jnp.where(pred, lo, mid + 1)
        return lo, hi

    lo, hi = lax.fori_loop(0, 31, body, (lo, hi))
    t = lo

    cnt_lt = jnp.sum((bits < t).astype(jnp.float32), axis=1, keepdims=True)
    eq = bits == t

    row_ids = lax.broadcasted_iota(jnp.int32, (_L, _L), 0)
    col_ids = lax.broadcasted_iota(jnp.int32, (_L, _L), 1)
    tri = (row_ids < col_ids).astype(jnp.float32)
    eq_rank = jnp.dot(eq.astype(jnp.float32), tri,
                      preferred_element_type=jnp.float32)

    keep = (bits < t) | (eq & (cnt_lt + eq_rank < float(len_keep)))
    keepf = keep.astype(jnp.float32)
    keep_ref[...] = keepf
    mask_ref[...] = 1.0 - keepf

    for c in range(_NCHUNKS):
        wait_load(c)
        b = c % _NBUF
        for i in range(_CHUNK_ROWS):
            row = c * _CHUNK_ROWS + i
            bufs_ref[b, i] = bufs_ref[b, i] * keep_ref[row:row + 1]
        start_store(c)
        nc = c + _NBUF
        if nc < _NCHUNKS:
            wait_store(nc - _NBUF)
            start_load(nc)
    for c in range(_NCHUNKS - _NBUF, _NCHUNKS):
        wait_store(c)


@jax.jit
def kernel(x):
    bs, nvars, L, D = x.shape
    len_keep = int(L * (1 - _MASK_RATIO))
    rows = bs * nvars

    noise = jax.random.uniform(jax.random.key(42), (bs, nvars, L),
                               dtype=jnp.float32)
    bits = lax.bitcast_convert_type(noise, jnp.int32).reshape(rows, L)

    xt = x.transpose(0, 1, 3, 2).reshape(rows, D, L)

    out, maskf = pl.pallas_call(
        functools.partial(_masked_fill_kernel, len_keep=len_keep),
        in_specs=[
            pl.BlockSpec((rows, L), lambda: (0, 0)),
            pl.BlockSpec(memory_space=pl.ANY),
        ],
        out_specs=[
            pl.BlockSpec(memory_space=pl.ANY),
            pl.BlockSpec((rows, L), lambda: (0, 0)),
        ],
        out_shape=[
            jax.ShapeDtypeStruct((rows, D, L), x.dtype),
            jax.ShapeDtypeStruct((rows, L), jnp.float32),
        ],
        scratch_shapes=[
            pltpu.VMEM((rows, L), jnp.float32),
            pltpu.VMEM((_NBUF, _CHUNK_ROWS, D, L), jnp.float32),
            pltpu.SemaphoreType.DMA((_NBUF,)),
            pltpu.SemaphoreType.DMA((_NBUF,)),
        ],
    )(bits, xt)

    x_mask = out.reshape(bs, nvars, D, L).transpose(0, 1, 3, 2)
    mask = maskf.reshape(bs, nvars, L).astype(bool)
    return (x_mask, mask)
